# baseline (speedup 1.0000x reference)
import functools

import jax
import jax.numpy as jnp
from jax import lax
from jax.experimental import pallas as pl
from jax.experimental.pallas import tpu as pltpu
from jax.experimental.pallas import tpu_sc as plsc

B = 4
N = 4096
D = 4096
K = N // 2
ROWS = B * K

NC = 2
NS = 16
NW = NC * NS
RPW = ROWS // NW
C = 8
NCHUNK = RPW // C

_JB = 128
_PB = 128


def _topk_body(rows_ref, cols_ref, vals_ref, idx_ref, gid_ref):
    i_row = lax.broadcasted_iota(jnp.int32, (1, N), 1)
    for b in range(B):
        row = rows_ref[b:b + 1, :]

        def rank_step(jc, rank):
            j0 = jc * _JB
            cj = cols_ref[pl.ds(j0, _JB), b:b + 1]
            jids = lax.broadcasted_iota(jnp.int32, (_JB, 1), 0) + j0
            beats = (cj > row) | ((cj == row) & (jids < i_row))
            return rank + jnp.sum(beats.astype(jnp.int32), axis=0,
                                  keepdims=True)

        rank = lax.fori_loop(0, N // _JB, rank_step,
                             jnp.zeros((1, N), jnp.int32))

        def inv_step(pc, carry):
            p0 = pc * _PB
            pids = lax.broadcasted_iota(jnp.int32, (_PB, 1), 0) + p0
            oh = rank == pids
            v = jnp.sum(jnp.where(oh, row, 0.0), axis=1, keepdims=True)
            ii = jnp.sum(jnp.where(oh, i_row, 0), axis=1, keepdims=True)
            vals_ref[pl.ds(p0, _PB), b:b + 1] = v
            idx_ref[pl.ds(p0, _PB), b:b + 1] = ii
            gid_ref[pl.ds(p0, _PB), b:b + 1] = ii + b * N
            return carry

        lax.fori_loop(0, K // _PB, inv_step, 0)


def _topk(scores):
    scores_t = scores.T
    vals_t, idx_t, gid_t = pl.pallas_call(
        _topk_body,
        out_shape=[
            jax.ShapeDtypeStruct((K, B), jnp.float32),
            jax.ShapeDtypeStruct((K, B), jnp.int32),
            jax.ShapeDtypeStruct((K, B), jnp.int32),
        ],
    )(scores, scores_t)
    return vals_t.T, idx_t.T, gid_t.T


def _gather_body(gid_hbm, hs_hbm, out_hbm, idx_v, buf0, buf1, g0, g1, w0, w1):
    wid = lax.axis_index("s") * NC + lax.axis_index("c")
    base = wid * RPW
    pltpu.sync_copy(gid_hbm.at[pl.ds(base, RPW)], idx_v)

    bufs = (buf0, buf1)
    gsems = (g0, g1)
    wsems = (w0, w1)

    def start_gather(c, bslot):
        pltpu.async_copy(
            hs_hbm.at[idx_v.at[pl.ds(c * C, C)]], bufs[bslot], gsems[bslot])

    def wait_gather(c, bslot):
        pltpu.make_async_copy(
            hs_hbm.at[idx_v.at[pl.ds(c * C, C)]], bufs[bslot],
            gsems[bslot]).wait()

    def start_write(c, bslot):
        pltpu.async_copy(
            bufs[bslot], out_hbm.at[pl.ds(base + c * C, C)], wsems[bslot])

    def wait_write(c, bslot):
        pltpu.make_async_copy(
            bufs[bslot], out_hbm.at[pl.ds(base + c * C, C)],
            wsems[bslot]).wait()

    start_gather(0, 0)
    start_gather(1, 1)

    def loop_body(it, _):
        c0 = it * 2
        for bslot in (0, 1):
            c = c0 + bslot
            wait_gather(c, bslot)
            start_write(c, bslot)
            wait_write(c, bslot)
            start_gather(c + 2, bslot)
        return _

    lax.fori_loop(0, (NCHUNK - 2) // 2, loop_body, None)
    for bslot in (0, 1):
        c = NCHUNK - 2 + bslot
        wait_gather(c, bslot)
        start_write(c, bslot)
        wait_write(c, bslot)


def _gather(hs_flat, gids):
    mesh = plsc.VectorSubcoreMesh(
        core_axis_name="c", subcore_axis_name="s", num_cores=NC,
        num_subcores=NS)
    run = pl.kernel(
        _gather_body,
        out_type=jax.ShapeDtypeStruct((ROWS, D), jnp.float32),
        mesh=mesh,
        scratch_types=[
            pltpu.VMEM((RPW,), jnp.int32),
            pltpu.VMEM((C, D), jnp.float32),
            pltpu.VMEM((C, D), jnp.float32),
            pltpu.SemaphoreType.DMA,
            pltpu.SemaphoreType.DMA,
            pltpu.SemaphoreType.DMA,
            pltpu.SemaphoreType.DMA,
        ],
    )
    return run(gids, hs_flat)


def kernel(scores, hidden_states):
    vals, idx, gid = _topk(scores)
    gids = gid.reshape(-1)
    selected = _gather(hidden_states.reshape(B * N, D), gids)
    batch_idx = jnp.broadcast_to(
        jnp.arange(B, dtype=jnp.int32)[:, None], (B, K)).reshape(-1)
    return (selected, batch_idx, idx.reshape(-1), vals.reshape(-1))


# speedup vs baseline: 2.0908x; 2.0908x over previous
import functools

import jax
import jax.numpy as jnp
from jax import lax
from jax.experimental import pallas as pl
from jax.experimental.pallas import tpu as pltpu
from jax.experimental.pallas import tpu_sc as plsc

B = 4
N = 4096
D = 4096
K = N // 2
ROWS = B * K

NC = 2
NS = 16
NW = NC * NS
RPW = ROWS // NW
C = 8
NCHUNK = RPW // C


_Q = N // 128
_KQ = K // 128


def _swap_dist(x, d):
    if d < 128:
        left = jnp.concatenate([x[:, d:], x[:, :d]], axis=1)
        right = jnp.concatenate([x[:, -d:], x[:, :-d]], axis=1)
        lane = lax.broadcasted_iota(jnp.int32, x.shape, 1)
        return jnp.where((lane & d) == 0, left, right)
    dq = d // 128
    left = jnp.concatenate([x[dq:, :], x[:dq, :]], axis=0)
    right = jnp.concatenate([x[-dq:, :], x[:-dq, :]], axis=0)
    sub = lax.broadcasted_iota(jnp.int32, x.shape, 0)
    return jnp.where((sub & dq) == 0, left, right)


def _topk_body(s_ref, vals_ref, idx_ref, gid_ref):
    v = s_ref[...]
    u = lax.bitcast_convert_type(v, jnp.int32)
    key = jnp.where(u < 0, u ^ jnp.int32(0x7FFFFFFF), u)
    sub = lax.broadcasted_iota(jnp.int32, (128, 128), 0)
    lane = lax.broadcasted_iota(jnp.int32, (128, 128), 1)
    elem = (sub % _Q) * 128 + lane
    idx = elem

    for L in range(1, 13):
        kbit = 1 << L
        for d in (1 << p for p in range(L - 1, -1, -1)):
            kp = _swap_dist(key, d)
            ip = _swap_dist(idx, d)
            less = (key > kp) | ((key == kp) & (idx < ip))
            if kbit == N * 2:
                take_hi = (elem & d) != 0
            else:
                take_hi = ((elem & kbit) != 0) ^ ((elem & d) != 0)
            cond = less ^ take_hi
            key = jnp.where(cond, key, kp)
            idx = jnp.where(cond, idx, ip)

    uo = jnp.where(key < 0, key ^ jnp.int32(0x7FFFFFFF), key)
    vo = lax.bitcast_convert_type(uo, jnp.float32)
    for r in range(B):
        vals_ref[r * _KQ:(r + 1) * _KQ, :] = vo[r * _Q:r * _Q + _KQ, :]
        ii = idx[r * _Q:r * _Q + _KQ, :]
        idx_ref[r * _KQ:(r + 1) * _KQ, :] = ii
        gid_ref[r * _KQ:(r + 1) * _KQ, :] = ii + r * N


def _topk(scores):
    vals, idx, gid = pl.pallas_call(
        _topk_body,
        out_shape=[
            jax.ShapeDtypeStruct((B * _KQ, 128), jnp.float32),
            jax.ShapeDtypeStruct((B * _KQ, 128), jnp.int32),
            jax.ShapeDtypeStruct((B * _KQ, 128), jnp.int32),
        ],
    )(scores.reshape(128, 128))
    return (vals.reshape(B, K), idx.reshape(B, K), gid.reshape(B, K))


def _gather_body(gid_hbm, hs_hbm, out_hbm, idx_v, buf0, buf1, g0, g1, w0, w1):
    wid = lax.axis_index("s") * NC + lax.axis_index("c")
    base = wid * RPW
    pltpu.sync_copy(gid_hbm.at[pl.ds(base, RPW)], idx_v)

    bufs = (buf0, buf1)
    gsems = (g0, g1)
    wsems = (w0, w1)

    def start_gather(c, bslot):
        pltpu.async_copy(
            hs_hbm.at[idx_v.at[pl.ds(c * C, C)]], bufs[bslot], gsems[bslot])

    def wait_gather(c, bslot):
        pltpu.make_async_copy(
            hs_hbm.at[idx_v.at[pl.ds(c * C, C)]], bufs[bslot],
            gsems[bslot]).wait()

    def start_write(c, bslot):
        pltpu.async_copy(
            bufs[bslot], out_hbm.at[pl.ds(base + c * C, C)], wsems[bslot])

    def wait_write(c, bslot):
        pltpu.make_async_copy(
            bufs[bslot], out_hbm.at[pl.ds(base + c * C, C)],
            wsems[bslot]).wait()

    start_gather(0, 0)
    start_gather(1, 1)

    def loop_body(it, _):
        c0 = it * 2
        for bslot in (0, 1):
            c = c0 + bslot
            wait_gather(c, bslot)
            start_write(c, bslot)
            wait_write(c, bslot)
            start_gather(c + 2, bslot)
        return _

    lax.fori_loop(0, (NCHUNK - 2) // 2, loop_body, None)
    for bslot in (0, 1):
        c = NCHUNK - 2 + bslot
        wait_gather(c, bslot)
        start_write(c, bslot)
        wait_write(c, bslot)


def _gather(hs_flat, gids):
    mesh = plsc.VectorSubcoreMesh(
        core_axis_name="c", subcore_axis_name="s", num_cores=NC,
        num_subcores=NS)
    run = pl.kernel(
        _gather_body,
        out_type=jax.ShapeDtypeStruct((ROWS, D), jnp.float32),
        mesh=mesh,
        scratch_types=[
            pltpu.VMEM((RPW,), jnp.int32),
            pltpu.VMEM((C, D), jnp.float32),
            pltpu.VMEM((C, D), jnp.float32),
            pltpu.SemaphoreType.DMA,
            pltpu.SemaphoreType.DMA,
            pltpu.SemaphoreType.DMA,
            pltpu.SemaphoreType.DMA,
        ],
    )
    return run(gids, hs_flat)


def kernel(scores, hidden_states):
    vals, idx, gid = _topk(scores)
    gids = gid.reshape(-1)
    selected = _gather(hidden_states.reshape(B * N, D), gids)
    batch_idx = jnp.broadcast_to(
        jnp.arange(B, dtype=jnp.int32)[:, None], (B, K)).reshape(-1)
    return (selected, batch_idx, idx.reshape(-1), vals.reshape(-1))
